# baseline (speedup 1.0000x reference)
import functools

import jax
import jax.numpy as jnp
from jax import lax
from jax.experimental import pallas as pl
from jax.experimental.pallas import tpu as pltpu
from jax.experimental.pallas import tpu_sc as plsc

B = 16384
D = 64
L = 16
NC = 2
NS = 16
NW = NC * NS
BPW = B // NW
CH = 128
HALF = 256
NCH = HALF // CH
NG = HALF // L

_mesh = plsc.VectorSubcoreMesh(core_axis_name="c", subcore_axis_name="s")


@functools.partial(
    pl.kernel,
    mesh=_mesh,
    compiler_params=pltpu.CompilerParams(
        needs_layout_passes=False,
        use_tc_tiling_on_sc=True,
        disable_bounds_checks=True,
    ),
    out_type=jax.ShapeDtypeStruct((B,), jnp.float32),
    scratch_types=[
        pltpu.VMEM((BPW,), jnp.int32),
        pltpu.VMEM((BPW,), jnp.int32),
        pltpu.VMEM((BPW,), jnp.int32),
        pltpu.VMEM((NCH, CH), jnp.int32),
        pltpu.VMEM((NCH, CH), jnp.int32),
        pltpu.VMEM((NCH, CH), jnp.int32),
        pltpu.VMEM((HALF, 128), jnp.float32),
        pltpu.VMEM((HALF, 128), jnp.float32),
        pltpu.VMEM((HALF, 128), jnp.float32),
        pltpu.VMEM((BPW,), jnp.float32),
        pltpu.SemaphoreType.DMA,
    ],
)
def _transe_sc(hidx_hbm, ridx_hbm, tidx_hbm, ent2_hbm, rel2_hbm, out_hbm,
               hs_v, rs_v, ts_v, hk_v, rk_v, tk_v, hD_v, rD_v, tD_v,
               out_v, sem):
    wid = lax.axis_index("s") * NC + lax.axis_index("c")
    base = wid * BPW

    pltpu.sync_copy(hidx_hbm.at[pl.ds(base, BPW)], hs_v)
    pltpu.sync_copy(ridx_hbm.at[pl.ds(base, BPW)], rs_v)
    pltpu.sync_copy(tidx_hbm.at[pl.ds(base, BPW)], ts_v)

    lane = lax.iota(jnp.int32, L)

    for p in range(BPW // HALF):
        def i_body(g, carry):
            col = g * L
            off = p * HALF + col
            for st_v, k_v in ((hs_v, hk_v), (rs_v, rk_v), (ts_v, tk_v)):
                k_v[g >> 3, pl.ds((g & 7) * L, L)] = st_v[pl.ds(off, L)] >> 1
            return carry

        lax.fori_loop(0, NG, i_body, 0)

        cps = []
        for c in range(NCH):
            dst = pl.ds(c * CH, CH)
            cps.append(pltpu.async_copy(ent2_hbm.at[hk_v.at[c]], hD_v.at[dst], sem))
            cps.append(pltpu.async_copy(rel2_hbm.at[rk_v.at[c]], rD_v.at[dst], sem))
            cps.append(pltpu.async_copy(ent2_hbm.at[tk_v.at[c]], tD_v.at[dst], sem))
        for cp in cps:
            cp.wait()

        def c_body(g, carry):
            col = g * L
            off = p * HALF + col
            slots = col + lane
            hc = (hs_v[pl.ds(off, L)] & 1) * D
            rc = (rs_v[pl.ds(off, L)] & 1) * D
            tc = (ts_v[pl.ds(off, L)] & 1) * D

            def d_body(d, acc):
                hv = plsc.load_gather(hD_v, [slots, hc + d])
                rv = plsc.load_gather(rD_v, [slots, rc + d])
                tv = plsc.load_gather(tD_v, [slots, tc + d])
                return acc + jnp.abs(hv + rv - tv)

            acc = lax.fori_loop(0, D, d_body, jnp.zeros((L,), jnp.float32))
            out_v[pl.ds(off, L)] = acc
            return carry

        lax.fori_loop(0, NG, c_body, 0)

    pltpu.sync_copy(out_v, out_hbm.at[pl.ds(base, BPW)])


def kernel(triples, ent, rel):
    tr = triples.astype(jnp.int32)
    e2 = ent.reshape(ent.shape[0] // 2, 2 * ent.shape[1])
    r2 = rel.reshape(rel.shape[0] // 2, 2 * rel.shape[1])
    return _transe_sc(tr[:, 0], tr[:, 1], tr[:, 2], e2, r2)
